# baseline (speedup 1.0000x reference)
import jax
import jax.numpy as jnp
from jax import lax
from jax.experimental import pallas as pl
from jax.experimental.pallas import tpu as pltpu
from jax.experimental.pallas import tpu_sc as plsc
from jax._src.pallas import mpmd as _pl_mpmd

B = 16384
C = 1000
NC = 2
NS = 16
NW = NC * NS
RPW = B // NW
ZBLK = 512
TBL = 16384


def _zero_body(t_ref, o_ref):
    o_ref[...] = jnp.zeros_like(o_ref)


_zero_fill = pl.pallas_call(
    _zero_body,
    out_shape=jax.ShapeDtypeStruct((B, C), jnp.float32),
    grid=(B // ZBLK,),
    in_specs=[pl.BlockSpec(memory_space=pl.ANY)],
    out_specs=pl.BlockSpec((ZBLK, C), lambda i: (i, 0)),
)


def _ones_body(tgt_hbm, zeros_hbm, out_hbm, idx_v, table_v, drain_v, sem):
    del zeros_hbm
    cid = lax.axis_index("c")
    sid = lax.axis_index("s")
    wid = sid * NC + cid
    base = pl.multiple_of(wid * RPW, 8)

    lanes = lax.iota(jnp.int32, 16)
    for r in range(8):
        table_v[pl.ds(1024 + 2048 * r, 16)] = \
            jnp.where(lanes == r, 1.0, 0.0)

    pltpu.sync_copy(tgt_hbm.at[pl.ds(base, RPW)], idx_v)

    @pl.loop(0, RPW // 16)
    def _grp(g):
        c16 = idx_v[pl.ds(pl.multiple_of(g * 16, 16), 16)]
        o16 = 1024 + jnp.bitwise_and(c16, 7) * 2048
        cb16 = jnp.bitwise_and(c16, ~7)
        for k in range(16):
            o = pl.multiple_of(o16[k], 8)
            cb = pl.multiple_of(cb16[k], 8)
            row = base + g * 16 + k
            pltpu.async_copy(table_v.at[pl.ds(o, 8)],
                             out_hbm.at[row, pl.ds(cb, 8)], sem)

    pltpu.make_async_copy(tgt_hbm.at[pl.ds(0, 4096)], drain_v, sem).wait()


_sc_mesh = plsc.VectorSubcoreMesh(core_axis_name="c", subcore_axis_name="s")

_sc_ones = _pl_mpmd._mpmd_map(
    [(_sc_mesh, _ones_body)],
    jax.ShapeDtypeStruct((B, C), jnp.float32),
    input_output_aliases={1: 0},
    scratch_types=[
        pltpu.VMEM((RPW,), jnp.int32),
        pltpu.VMEM((TBL,), jnp.float32),
        pltpu.VMEM((4096,), jnp.int32),
        pltpu.SemaphoreType.DMA,
    ],
    compiler_params=pltpu.CompilerParams(needs_layout_passes=False),
    interpret=False,
    debug=False,
    cost_estimate=None,
    name="sc_one_hot_scatter",
    metadata=None,
)


def kernel(target):
    tgt = target.astype(jnp.int32)
    return _sc_ones(tgt, _zero_fill(tgt))


# speedup vs baseline: 2.6087x; 2.6087x over previous
import jax
import jax.numpy as jnp
from jax import lax
from jax.experimental import pallas as pl
from jax.experimental.pallas import tpu as pltpu
from jax.experimental.pallas import tpu_sc as plsc
from jax._src.pallas import mpmd as _pl_mpmd

B = 16384
C = 1000
NC = 2
NS = 16
NW = NC * NS
RPW = B // NW
ZBLK = 1024
TBL = 16384


def _zero_body(t_ref, o_ref):
    o_ref[...] = jnp.zeros_like(o_ref)


_zero_fill = pl.pallas_call(
    _zero_body,
    out_shape=jax.ShapeDtypeStruct((C, B), jnp.float32),
    grid=(B // ZBLK,),
    in_specs=[pl.BlockSpec(memory_space=pl.ANY)],
    out_specs=pl.BlockSpec((C, ZBLK), lambda i: (0, i)),
)


def _ones_body(tgt_hbm, zeros_hbm, out_hbm, idx_v, table_v, drain_v, sem):
    del zeros_hbm
    cid = lax.axis_index("c")
    sid = lax.axis_index("s")
    wid = sid * NC + cid
    base = pl.multiple_of(wid * RPW, 8)

    lanes = lax.iota(jnp.int32, 16)
    for r in range(8):
        table_v[pl.ds(1024 + 2048 * r, 16)] = \
            jnp.where(lanes == r, 1.0, 0.0)

    pltpu.sync_copy(tgt_hbm.at[pl.ds(base, RPW)], idx_v)

    @pl.loop(0, RPW // 16)
    def _grp(g):
        c16 = idx_v[pl.ds(pl.multiple_of(g * 16, 16), 16)]
        for k in range(16):
            c = c16[k]
            col = pl.multiple_of(base + g * 16 + (k & ~7), 8)
            src = table_v.at[pl.ds(1024 + 2048 * (k & 7), 8)]
            pltpu.async_copy(src, out_hbm.at[c, pl.ds(col, 8)], sem)

    pltpu.make_async_copy(tgt_hbm.at[pl.ds(0, 4096)], drain_v, sem).wait()


_sc_mesh = plsc.VectorSubcoreMesh(core_axis_name="c", subcore_axis_name="s")

_sc_ones = _pl_mpmd._mpmd_map(
    [(_sc_mesh, _ones_body)],
    jax.ShapeDtypeStruct((C, B), jnp.float32),
    input_output_aliases={1: 0},
    scratch_types=[
        pltpu.VMEM((RPW,), jnp.int32),
        pltpu.VMEM((TBL,), jnp.float32),
        pltpu.VMEM((4096,), jnp.int32),
        pltpu.SemaphoreType.DMA,
    ],
    compiler_params=pltpu.CompilerParams(needs_layout_passes=False),
    interpret=False,
    debug=False,
    cost_estimate=None,
    name="sc_one_hot_scatter",
    metadata=None,
)


def kernel(target):
    tgt = target.astype(jnp.int32)
    return _sc_ones(tgt, _zero_fill(tgt)).T
